# baseline (speedup 1.0000x reference)
import functools

import jax
import jax.numpy as jnp
from jax import lax
from jax.experimental import pallas as pl
from jax.experimental.pallas import tpu as pltpu
from jax.experimental.pallas import tpu_sc as plsc

N = 50000
E = 800000
G = 64
NP = 50176
BLK = 512
NBLK = NP // BLK
EP = 819200
EW = EP // 32
ECH = EW // 128
ET = EP // 16
NB = 12544
ACC2 = 12800
DUMP = 12544
NC, NS = 2, 16

_MESH = dict(core_axis_name="c", subcore_axis_name="s", num_cores=NC,
             num_subcores=NS)


def _sc_deg(dst2d):
    @functools.partial(
        pl.kernel,
        out_type=jax.ShapeDtypeStruct((NC * NP,), jnp.float32),
        mesh=plsc.VectorSubcoreMesh(**_MESH),
        scratch_types=[
            pltpu.VMEM((ECH, 128), jnp.int32),
            pltpu.VMEM((128,), jnp.float32),
            pltpu.VMEM((392,), jnp.float32),
            pltpu.VMEM_SHARED((NP,), jnp.float32),
            pltpu.SemaphoreType.DMA,
            pltpu.SemaphoreType.DMA,
        ],
    )
    def k(dst_h, out_h, dstbuf, ones, zbuf, acc, sem0, sem1):
        c = lax.axis_index("c")
        s = lax.axis_index("s")
        w = c * NS + s

        def zb(i, _):
            zbuf[pl.ds(i * 16, 16)] = jnp.zeros((16,), jnp.float32)
            return 0

        lax.fori_loop(0, 392 // 16, zb, 0)

        def zc(j, _):
            pltpu.sync_copy(zbuf, acc.at[pl.ds(s * 3136 + j * 392, 392)])
            return 0

        lax.fori_loop(0, 8, zc, 0)
        for v in range(8):
            ones[pl.ds(v * 16, 16)] = jnp.ones((16,), jnp.float32)
        pltpu.sync_copy(dst_h.at[pl.ds(w * ECH, ECH)], dstbuf)
        plsc.subcore_barrier()

        def body(i, _):
            d0 = pltpu.async_copy(ones, acc.at[dstbuf.at[i * 2]], sem0,
                                  add=True)
            d1 = pltpu.async_copy(ones, acc.at[dstbuf.at[i * 2 + 1]], sem1,
                                  add=True)
            d0.wait()
            d1.wait()
            return 0

        lax.fori_loop(0, ECH // 2, body, 0)
        plsc.subcore_barrier()

        def co(j, _):
            pltpu.sync_copy(acc.at[pl.ds(s * 3136 + j * 392, 392)], zbuf)
            pltpu.sync_copy(zbuf,
                            out_h.at[pl.ds(c * NP + s * 3136 + j * 392, 392)])
            return 0

        lax.fori_loop(0, 8, co, 0)

    return k(dst2d)


def _tc_prep(dega, degb, xp):
    def body(da_ref, db_ref, x_ref, dinv_ref, u1_ref):
        deg = da_ref[...] + db_ref[...] + 1.0
        dv = lax.rsqrt(deg)
        dinv_ref[...] = dv
        u1_ref[...] = x_ref[...] * dv

    return pl.pallas_call(
        body,
        grid=(NBLK,),
        in_specs=[
            pl.BlockSpec((BLK, 1), lambda i: (i, 0)),
            pl.BlockSpec((BLK, 1), lambda i: (i, 0)),
            pl.BlockSpec((BLK, 128), lambda i: (i, 0)),
        ],
        out_specs=[
            pl.BlockSpec((BLK, 1), lambda i: (i, 0)),
            pl.BlockSpec((BLK, 128), lambda i: (i, 0)),
        ],
        out_shape=[
            jax.ShapeDtypeStruct((NP, 1), jnp.float32),
            jax.ShapeDtypeStruct((NP, 128), jnp.float32),
        ],
    )(dega, degb, xp)


def _tc_layer1(s1, u1p, dinv, Wg1p, bg1r):
    def body(s_ref, u_ref, dv_ref, w_ref, bias_ref, out_ref):
        dv = dv_ref[...]
        agg = dv * (s_ref[...] + u_ref[...])
        h = jnp.dot(agg, w_ref[...], preferred_element_type=jnp.float32)
        h = jnp.maximum(h + bias_ref[...], 0.0)
        out_ref[...] = dv * h

    return pl.pallas_call(
        body,
        grid=(NBLK,),
        in_specs=[
            pl.BlockSpec((BLK, 128), lambda i: (i, 0)),
            pl.BlockSpec((BLK, 128), lambda i: (i, 0)),
            pl.BlockSpec((BLK, 1), lambda i: (i, 0)),
            pl.BlockSpec((128, 128), lambda i: (0, 0)),
            pl.BlockSpec((1, 128), lambda i: (0, 0)),
        ],
        out_specs=pl.BlockSpec((BLK, 128), lambda i: (i, 0)),
        out_shape=jax.ShapeDtypeStruct((NP, 128), jnp.float32),
    )(s1, u1p, dinv, Wg1p, bg1r)


CAP = 51328
STAGE = 2144


def _sc_compact(src1d, dst1d):
    @functools.partial(
        pl.kernel,
        out_type=(
            jax.ShapeDtypeStruct((32 * 2 * CAP,), jnp.int32),
            jax.ShapeDtypeStruct((32 * 2 * 16,), jnp.int32),
        ),
        mesh=plsc.VectorSubcoreMesh(**_MESH),
        scratch_types=[
            pltpu.VMEM((2048,), jnp.int32),
            pltpu.VMEM((2048,), jnp.int32),
            pltpu.VMEM((STAGE,), jnp.int32),
            pltpu.VMEM((STAGE,), jnp.int32),
            pltpu.VMEM((16,), jnp.int32),
        ],
    )
    def k(src_h, dst_h, stash_h, counts_h, sbuf, dbuf, st0, st1, cvb):
        c = lax.axis_index("c")
        s = lax.axis_index("s")
        w = c * NS + s
        lo0 = (c * 2) * NB
        lo1 = lo0 + NB
        hi1 = lo1 + NB
        b0 = (w * 2) * CAP
        b1 = (w * 2 + 1) * CAP
        dump = jnp.full((16,), DUMP << 16, jnp.int32)

        def stripe(t, carry):
            cc0, cc1 = carry
            base = s * ET + t * 2048
            pltpu.sync_copy(src_h.at[pl.ds(base, 2048)], sbuf)
            pltpu.sync_copy(dst_h.at[pl.ds(base, 2048)], dbuf)

            def fb(i, cnts):
                cnt0, cnt1 = cnts
                sv = sbuf[pl.ds(i * 16, 16)]
                dv = dbuf[pl.ds(i * 16, 16)]
                m0 = (dv >= lo0) & (dv < lo1)
                m1 = (dv >= lo1) & (dv < hi1)
                p0 = sv | jnp.where(m0, dv - lo0, DUMP) << 16
                p1 = sv | jnp.where(m1, dv - lo1, DUMP) << 16
                for j in range(16):
                    st0[pl.ds(cnt0, 16)] = (jnp.zeros((16,), jnp.int32)
                                            + p0[j])
                    st1[pl.ds(cnt1, 16)] = (jnp.zeros((16,), jnp.int32)
                                            + p1[j])
                    dj = dv[j]
                    cnt0 = cnt0 + jnp.where((dj >= lo0) & (dj < lo1), 1, 0)
                    cnt1 = cnt1 + jnp.where((dj >= lo1) & (dj < hi1), 1, 0)
                return cnt0, cnt1

            cnt0, cnt1 = lax.fori_loop(0, 128, fb, (0, 0))
            for v in range(4):
                st0[pl.ds(cnt0 + v * 16, 16)] = dump
                st1[pl.ds(cnt1 + v * 16, 16)] = dump
            nf0 = (cnt0 + 63) // 64
            nf1 = (cnt1 + 63) // 64

            def fl0(j, _):
                pltpu.sync_copy(
                    st0.at[pl.ds(j * 64, 64)],
                    stash_h.at[pl.ds(b0 + (cc0 + j) * 64, 64)])
                return 0

            def fl1(j, _):
                pltpu.sync_copy(
                    st1.at[pl.ds(j * 64, 64)],
                    stash_h.at[pl.ds(b1 + (cc1 + j) * 64, 64)])
                return 0

            lax.fori_loop(0, nf0, fl0, 0)
            lax.fori_loop(0, nf1, fl1, 0)
            return cc0 + nf0, cc1 + nf1

        cc0, cc1 = lax.fori_loop(0, 25, stripe, (0, 0))
        for v in range(4):
            st0[pl.ds(v * 16, 16)] = dump
        for extra in range(2):
            pltpu.sync_copy(st0.at[pl.ds(0, 64)],
                            stash_h.at[pl.ds(b0 + (cc0 + extra) * 64, 64)])
            pltpu.sync_copy(st0.at[pl.ds(0, 64)],
                            stash_h.at[pl.ds(b1 + (cc1 + extra) * 64, 64)])
        cc0 = jnp.where(cc0 == 0, 2, ((cc0 + 1) // 2) * 2)
        cc1 = jnp.where(cc1 == 0, 2, ((cc1 + 1) // 2) * 2)
        cvb[pl.ds(0, 16)] = jnp.zeros((16,), jnp.int32) + cc0
        pltpu.sync_copy(cvb, counts_h.at[pl.ds(w * 32, 16)])
        cvb[pl.ds(0, 16)] = jnp.zeros((16,), jnp.int32) + cc1
        pltpu.sync_copy(cvb, counts_h.at[pl.ds(w * 32 + 16, 16)])

    return k(src1d, dst1d)


def _sc_scatter(stash, counts, u):
    @functools.partial(
        pl.kernel,
        out_type=jax.ShapeDtypeStruct((NP, 128), jnp.float32),
        mesh=plsc.VectorSubcoreMesh(**_MESH),
        scratch_types=[
            pltpu.VMEM((64,), jnp.int32),
            pltpu.VMEM((64,), jnp.int32),
            pltpu.VMEM((64,), jnp.int32),
            pltpu.VMEM((64,), jnp.int32),
            pltpu.VMEM((64,), jnp.int32),
            pltpu.VMEM((64,), jnp.int32),
            pltpu.VMEM((64, 128), jnp.float32),
            pltpu.VMEM((64, 128), jnp.float32),
            pltpu.VMEM((8, 128), jnp.float32),
            pltpu.VMEM((16,), jnp.int32),
            pltpu.VMEM_SHARED((ACC2, 128), jnp.float32),
            pltpu.SemaphoreType.DMA,
            pltpu.SemaphoreType.DMA,
            pltpu.SemaphoreType.DMA,
            pltpu.SemaphoreType.DMA,
        ],
    )
    def k(stash_h, counts_h, u_h, out_h, pbuf0, pbuf1, schunk0, schunk1,
          dchunk0, dchunk1, rows0, rows1, zbuf, cvb, acc, sem0, sem1,
          sems0, sems1):
        c = lax.axis_index("c")
        s = lax.axis_index("s")
        w = c * NS + s

        def zb(i, _):
            for v in range(8):
                zbuf[i, pl.ds(v * 16, 16)] = jnp.zeros((16,), jnp.float32)
            return 0

        lax.fori_loop(0, 8, zb, 0)

        for kb in range(2):
            lo = (c * 2 + kb) * NB

            def zc(j, _):
                pltpu.sync_copy(zbuf, acc.at[pl.ds(s * 800 + j * 8, 8)])
                return 0

            lax.fori_loop(0, 100, zc, 0)
            plsc.subcore_barrier()
            pltpu.sync_copy(counts_h.at[pl.ds(w * 32 + kb * 16, 16)], cvb)
            nch = cvb[pl.ds(0, 16)][0]
            base = (w * 2 + kb) * CAP

            def load_pair(j0):
                pltpu.sync_copy(stash_h.at[pl.ds(base + j0 * 64, 64)], pbuf0)
                for v in range(4):
                    pv = pbuf0[pl.ds(v * 16, 16)]
                    schunk0[pl.ds(v * 16, 16)] = pv & 0xFFFF
                    dchunk0[pl.ds(v * 16, 16)] = pv >> 16
                d0 = pltpu.async_copy(u_h.at[schunk0], rows0, sem0)
                pltpu.sync_copy(stash_h.at[pl.ds(base + j0 * 64 + 64, 64)],
                                pbuf1)
                for v in range(4):
                    pv = pbuf1[pl.ds(v * 16, 16)]
                    schunk1[pl.ds(v * 16, 16)] = pv & 0xFFFF
                    dchunk1[pl.ds(v * 16, 16)] = pv >> 16
                d1 = pltpu.async_copy(u_h.at[schunk1], rows1, sem1)
                d0.wait()
                pltpu.async_copy(rows0, acc.at[dchunk0], sems0, add=True)
                d1.wait()
                pltpu.async_copy(rows1, acc.at[dchunk1], sems1, add=True)

            load_pair(0)

            def db(i, _):
                pltpu.make_async_copy(rows0, acc.at[dchunk0], sems0).wait()
                pltpu.make_async_copy(rows1, acc.at[dchunk1], sems1).wait()
                load_pair(i * 2)
                return 0

            lax.fori_loop(1, nch // 2, db, 0)
            pltpu.make_async_copy(rows0, acc.at[dchunk0], sems0).wait()
            pltpu.make_async_copy(rows1, acc.at[dchunk1], sems1).wait()
            plsc.subcore_barrier()

            def co(j, _):
                pltpu.sync_copy(acc.at[pl.ds(s * 784 + j * 56, 56)],
                                rows0.at[pl.ds(0, 56)])
                pltpu.sync_copy(rows0.at[pl.ds(0, 56)],
                                out_h.at[pl.ds(lo + s * 784 + j * 56, 56)])
                return 0

            lax.fori_loop(0, 14, co, 0)
            plsc.subcore_barrier()

    return k(stash, counts, u)


def _tc_final(s2, u2p, dinv, batch2d, Wg2, bg2r, W1, b1r, W2, b2r, W3, b3r,
              W4, b4r):
    neg = float("-inf")

    def body(s2_ref, u2_ref, dv_ref, bt_ref, wg_ref, bg_ref, w1_ref, c1_ref,
             w2_ref, c2_ref, w3_ref, c3_ref, w4_ref, c4_ref, out_ref, acc):
        i = pl.program_id(0)

        @pl.when(i == 0)
        def _():
            acc[...] = jnp.full((G, 128), neg, jnp.float32)

        dv = dv_ref[...]
        h2 = jnp.dot(dv * (s2_ref[...] + u2_ref[...]), wg_ref[...],
                     preferred_element_type=jnp.float32)
        h2 = jnp.maximum(h2 + bg_ref[...], 0.0)
        row = i * BLK + lax.broadcasted_iota(jnp.int32, (BLK, 1), 0)
        hm = jnp.where(row < N, h2, neg)
        bt = bt_ref[...]
        g_lo = bt_ref[0, 0]
        g_hi = bt_ref[BLK - 1, 0]
        gi = lax.broadcasted_iota(jnp.int32, (G, 1), 0)

        def body_g(g, _):
            v = jnp.max(jnp.where(bt == g, hm, neg), axis=0, keepdims=True)
            acc[...] = jnp.where(gi == g, jnp.maximum(acc[...], v), acc[...])
            return 0

        lax.fori_loop(g_lo, g_hi + 1, body_g, 0)

        @pl.when(i == NBLK - 1)
        def _():
            h = acc[...]
            h = jnp.maximum(jnp.dot(h, w1_ref[...],
                                    preferred_element_type=jnp.float32)
                            + c1_ref[...], 0.0)
            h = jnp.maximum(jnp.dot(h, w2_ref[...],
                                    preferred_element_type=jnp.float32)
                            + c2_ref[...], 0.0)
            h = jnp.maximum(jnp.dot(h, w3_ref[...],
                                    preferred_element_type=jnp.float32)
                            + c3_ref[...], 0.0)
            h = jnp.maximum(jnp.dot(h, w4_ref[...],
                                    preferred_element_type=jnp.float32)
                            + c4_ref[...], 0.0)
            out_ref[...] = h

    full = lambda shape: pl.BlockSpec(shape, lambda i: (0, 0))
    return pl.pallas_call(
        body,
        grid=(NBLK,),
        in_specs=[
            pl.BlockSpec((BLK, 128), lambda i: (i, 0)),
            pl.BlockSpec((BLK, 128), lambda i: (i, 0)),
            pl.BlockSpec((BLK, 1), lambda i: (i, 0)),
            pl.BlockSpec((BLK, 1), lambda i: (i, 0)),
            full((128, 128)), full((1, 128)),
            full((128, 128)), full((1, 128)),
            full((128, 64)), full((1, 64)),
            full((64, 32)), full((1, 32)),
            full((32, 16)), full((1, 16)),
        ],
        out_specs=pl.BlockSpec((G, 16), lambda i: (0, 0)),
        out_shape=jax.ShapeDtypeStruct((G, 16), jnp.float32),
        scratch_shapes=[pltpu.VMEM((G, 128), jnp.float32)],
    )(s2, u2p, dinv, batch2d, Wg2, bg2r, W1, b1r, W2, b2r, W3, b3r, W4, b4r)


def kernel(x, edge_index, batch, Wg1, bg1, Wg2, bg2, W1, b1, W2, b2, W3, b3,
           W4, b4):
    f32 = jnp.float32
    i32 = jnp.int32
    src = jnp.concatenate([edge_index[0], jnp.zeros((EP - E,), i32)])
    dst = jnp.concatenate([edge_index[1], jnp.full((EP - E,), NP - 1, i32)])
    src2d = src.reshape(EP // 128, 128)
    dst2d = dst.reshape(EP // 128, 128)
    xp = jnp.zeros((NP, 128), f32).at[:N, :8].set(x)
    batch2d = jnp.concatenate([batch, jnp.full((NP - N,), G - 1, i32)])
    batch2d = batch2d.reshape(NP, 1)
    Wg1p = jnp.zeros((128, 128), f32).at[:8].set(Wg1)

    degp = _sc_deg(dst2d)
    stash, counts = _sc_compact(src, dst)
    dinv, u1p = _tc_prep(degp[:NP].reshape(NP, 1), degp[NP:].reshape(NP, 1),
                         xp)
    s1 = _sc_scatter(stash, counts, u1p)
    u2p = _tc_layer1(s1, u1p, dinv, Wg1p, bg1.reshape(1, 128))
    s2 = _sc_scatter(stash, counts, u2p)
    return _tc_final(s2, u2p, dinv, batch2d, Wg2, bg2.reshape(1, 128),
                     W1, b1.reshape(1, 128), W2, b2.reshape(1, 64),
                     W3, b3.reshape(1, 32), W4, b4.reshape(1, 16))


# speedup vs baseline: 1.0394x; 1.0394x over previous
import functools

import jax
import jax.numpy as jnp
from jax import lax
from jax.experimental import pallas as pl
from jax.experimental.pallas import tpu as pltpu
from jax.experimental.pallas import tpu_sc as plsc

N = 50000
E = 800000
G = 64
NP = 50176
BLK = 512
NBLK = NP // BLK
EP = 819200
EW = EP // 32
ECH = EW // 128
ET = EP // 16
NB = 12544
ACC2 = 12800
DUMP = 12544
NC, NS = 2, 16

_MESH = dict(core_axis_name="c", subcore_axis_name="s", num_cores=NC,
             num_subcores=NS)


def _sc_deg(dst2d):
    @functools.partial(
        pl.kernel,
        out_type=jax.ShapeDtypeStruct((NC * NP,), jnp.float32),
        mesh=plsc.VectorSubcoreMesh(**_MESH),
        scratch_types=[
            pltpu.VMEM((ECH, 128), jnp.int32),
            pltpu.VMEM((128,), jnp.float32),
            pltpu.VMEM((392,), jnp.float32),
            pltpu.VMEM_SHARED((NP,), jnp.float32),
            pltpu.SemaphoreType.DMA,
            pltpu.SemaphoreType.DMA,
        ],
    )
    def k(dst_h, out_h, dstbuf, ones, zbuf, acc, sem0, sem1):
        c = lax.axis_index("c")
        s = lax.axis_index("s")
        w = c * NS + s

        def zb(i, _):
            zbuf[pl.ds(i * 16, 16)] = jnp.zeros((16,), jnp.float32)
            return 0

        lax.fori_loop(0, 392 // 16, zb, 0)

        def zc(j, _):
            pltpu.sync_copy(zbuf, acc.at[pl.ds(s * 3136 + j * 392, 392)])
            return 0

        lax.fori_loop(0, 8, zc, 0)
        for v in range(8):
            ones[pl.ds(v * 16, 16)] = jnp.ones((16,), jnp.float32)
        pltpu.sync_copy(dst_h.at[pl.ds(w * ECH, ECH)], dstbuf)
        plsc.subcore_barrier()

        def body(i, _):
            d0 = pltpu.async_copy(ones, acc.at[dstbuf.at[i * 2]], sem0,
                                  add=True)
            d1 = pltpu.async_copy(ones, acc.at[dstbuf.at[i * 2 + 1]], sem1,
                                  add=True)
            d0.wait()
            d1.wait()
            return 0

        lax.fori_loop(0, ECH // 2, body, 0)
        plsc.subcore_barrier()

        def co(j, _):
            pltpu.sync_copy(acc.at[pl.ds(s * 3136 + j * 392, 392)], zbuf)
            pltpu.sync_copy(zbuf,
                            out_h.at[pl.ds(c * NP + s * 3136 + j * 392, 392)])
            return 0

        lax.fori_loop(0, 8, co, 0)

    return k(dst2d)


def _tc_prep(dega, degb, xp):
    def body(da_ref, db_ref, x_ref, dinv_ref, u1_ref):
        deg = da_ref[...] + db_ref[...] + 1.0
        dv = lax.rsqrt(deg)
        dinv_ref[...] = dv
        u1_ref[...] = x_ref[...] * dv

    return pl.pallas_call(
        body,
        grid=(NBLK,),
        in_specs=[
            pl.BlockSpec((BLK, 1), lambda i: (i, 0)),
            pl.BlockSpec((BLK, 1), lambda i: (i, 0)),
            pl.BlockSpec((BLK, 128), lambda i: (i, 0)),
        ],
        out_specs=[
            pl.BlockSpec((BLK, 1), lambda i: (i, 0)),
            pl.BlockSpec((BLK, 128), lambda i: (i, 0)),
        ],
        out_shape=[
            jax.ShapeDtypeStruct((NP, 1), jnp.float32),
            jax.ShapeDtypeStruct((NP, 128), jnp.float32),
        ],
    )(dega, degb, xp)


def _tc_layer1(s1, u1p, dinv, Wg1p, bg1r):
    def body(s_ref, u_ref, dv_ref, w_ref, bias_ref, out_ref):
        dv = dv_ref[...]
        agg = dv * (s_ref[...] + u_ref[...])
        h = jnp.dot(agg, w_ref[...], preferred_element_type=jnp.float32)
        h = jnp.maximum(h + bias_ref[...], 0.0)
        out_ref[...] = dv * h

    return pl.pallas_call(
        body,
        grid=(NBLK,),
        in_specs=[
            pl.BlockSpec((BLK, 128), lambda i: (i, 0)),
            pl.BlockSpec((BLK, 128), lambda i: (i, 0)),
            pl.BlockSpec((BLK, 1), lambda i: (i, 0)),
            pl.BlockSpec((128, 128), lambda i: (0, 0)),
            pl.BlockSpec((1, 128), lambda i: (0, 0)),
        ],
        out_specs=pl.BlockSpec((BLK, 128), lambda i: (i, 0)),
        out_shape=jax.ShapeDtypeStruct((NP, 128), jnp.float32),
    )(s1, u1p, dinv, Wg1p, bg1r)


CAP = 51264
STAGE = 2144


def _sc_compact(src1d, dst1d):
    @functools.partial(
        pl.kernel,
        out_type=(
            jax.ShapeDtypeStruct((32 * 2 * CAP,), jnp.int32),
            jax.ShapeDtypeStruct((32 * 2 * 16,), jnp.int32),
        ),
        mesh=plsc.VectorSubcoreMesh(**_MESH),
        scratch_types=[
            pltpu.VMEM((2048,), jnp.int32),
            pltpu.VMEM((2048,), jnp.int32),
            pltpu.VMEM((STAGE,), jnp.int32),
            pltpu.VMEM((STAGE,), jnp.int32),
            pltpu.VMEM((16,), jnp.int32),
        ],
    )
    def k(src_h, dst_h, stash_h, counts_h, sbuf, dbuf, st0, st1, cvb):
        c = lax.axis_index("c")
        s = lax.axis_index("s")
        w = c * NS + s
        lo0 = (c * 2) * NB
        lo1 = lo0 + NB
        hi1 = lo1 + NB
        b0 = (w * 2) * CAP
        b1 = (w * 2 + 1) * CAP
        dump = jnp.full((16,), DUMP << 16, jnp.int32)

        def stripe(t, carry):
            cc0, cc1 = carry
            base = s * ET + t * 2048
            pltpu.sync_copy(src_h.at[pl.ds(base, 2048)], sbuf)
            pltpu.sync_copy(dst_h.at[pl.ds(base, 2048)], dbuf)

            def fb(i, cnts):
                cnt0, cnt1 = cnts
                sv = sbuf[pl.ds(i * 16, 16)]
                dv = dbuf[pl.ds(i * 16, 16)]
                m0 = (dv >= lo0) & (dv < lo1)
                m1 = (dv >= lo1) & (dv < hi1)
                p0 = sv | jnp.where(m0, dv - lo0, DUMP) << 16
                p1 = sv | jnp.where(m1, dv - lo1, DUMP) << 16
                for j in range(16):
                    st0[pl.ds(cnt0, 16)] = (jnp.zeros((16,), jnp.int32)
                                            + p0[j])
                    st1[pl.ds(cnt1, 16)] = (jnp.zeros((16,), jnp.int32)
                                            + p1[j])
                    dj = dv[j]
                    cnt0 = cnt0 + jnp.where((dj >= lo0) & (dj < lo1), 1, 0)
                    cnt1 = cnt1 + jnp.where((dj >= lo1) & (dj < hi1), 1, 0)
                return cnt0, cnt1

            cnt0, cnt1 = lax.fori_loop(0, 128, fb, (0, 0))
            for v in range(4):
                st0[pl.ds(cnt0 + v * 16, 16)] = dump
                st1[pl.ds(cnt1 + v * 16, 16)] = dump
            nf0 = (cnt0 + 63) // 64
            nf1 = (cnt1 + 63) // 64

            def fl0(j, _):
                pltpu.sync_copy(
                    st0.at[pl.ds(j * 64, 64)],
                    stash_h.at[pl.ds(b0 + (cc0 + j) * 64, 64)])
                return 0

            def fl1(j, _):
                pltpu.sync_copy(
                    st1.at[pl.ds(j * 64, 64)],
                    stash_h.at[pl.ds(b1 + (cc1 + j) * 64, 64)])
                return 0

            lax.fori_loop(0, nf0, fl0, 0)
            lax.fori_loop(0, nf1, fl1, 0)
            return cc0 + nf0, cc1 + nf1

        cc0, cc1 = lax.fori_loop(0, 25, stripe, (0, 0))
        for v in range(4):
            st0[pl.ds(v * 16, 16)] = dump
        pltpu.sync_copy(st0.at[pl.ds(0, 64)],
                        stash_h.at[pl.ds(b0 + cc0 * 64, 64)])
        pltpu.sync_copy(st0.at[pl.ds(0, 64)],
                        stash_h.at[pl.ds(b1 + cc1 * 64, 64)])
        cc0 = ((cc0 + 1) // 2) * 2
        cc1 = ((cc1 + 1) // 2) * 2
        cvb[pl.ds(0, 16)] = jnp.zeros((16,), jnp.int32) + cc0
        pltpu.sync_copy(cvb, counts_h.at[pl.ds(w * 32, 16)])
        cvb[pl.ds(0, 16)] = jnp.zeros((16,), jnp.int32) + cc1
        pltpu.sync_copy(cvb, counts_h.at[pl.ds(w * 32 + 16, 16)])

    return k(src1d, dst1d)


def _sc_scatter(stash, counts, u):
    @functools.partial(
        pl.kernel,
        out_type=jax.ShapeDtypeStruct((NP, 128), jnp.float32),
        mesh=plsc.VectorSubcoreMesh(**_MESH),
        scratch_types=[
            pltpu.VMEM((64,), jnp.int32),
            pltpu.VMEM((64,), jnp.int32),
            pltpu.VMEM((64,), jnp.int32),
            pltpu.VMEM((64,), jnp.int32),
            pltpu.VMEM((64,), jnp.int32),
            pltpu.VMEM((64,), jnp.int32),
            pltpu.VMEM((64, 128), jnp.float32),
            pltpu.VMEM((64, 128), jnp.float32),
            pltpu.VMEM((8, 128), jnp.float32),
            pltpu.VMEM((16,), jnp.int32),
            pltpu.VMEM_SHARED((ACC2, 128), jnp.float32),
            pltpu.SemaphoreType.DMA,
            pltpu.SemaphoreType.DMA,
        ],
    )
    def k(stash_h, counts_h, u_h, out_h, pbuf0, pbuf1, schunk0, schunk1,
          dchunk0, dchunk1, rows0, rows1, zbuf, cvb, acc, sem0, sem1):
        c = lax.axis_index("c")
        s = lax.axis_index("s")
        w = c * NS + s

        def zb(i, _):
            for v in range(8):
                zbuf[i, pl.ds(v * 16, 16)] = jnp.zeros((16,), jnp.float32)
            return 0

        lax.fori_loop(0, 8, zb, 0)

        for kb in range(2):
            lo = (c * 2 + kb) * NB

            def zc(j, _):
                pltpu.sync_copy(zbuf, acc.at[pl.ds(s * 800 + j * 8, 8)])
                return 0

            lax.fori_loop(0, 100, zc, 0)
            plsc.subcore_barrier()
            pltpu.sync_copy(counts_h.at[pl.ds(w * 32 + kb * 16, 16)], cvb)
            nch = cvb[pl.ds(0, 16)][0]
            base = (w * 2 + kb) * CAP

            def db(i, _):
                j0 = i * 2
                pltpu.sync_copy(stash_h.at[pl.ds(base + j0 * 64, 64)], pbuf0)
                for v in range(4):
                    pv = pbuf0[pl.ds(v * 16, 16)]
                    schunk0[pl.ds(v * 16, 16)] = pv & 0xFFFF
                    dchunk0[pl.ds(v * 16, 16)] = pv >> 16
                d0 = pltpu.async_copy(u_h.at[schunk0], rows0, sem0)
                pltpu.sync_copy(stash_h.at[pl.ds(base + j0 * 64 + 64, 64)],
                                pbuf1)
                for v in range(4):
                    pv = pbuf1[pl.ds(v * 16, 16)]
                    schunk1[pl.ds(v * 16, 16)] = pv & 0xFFFF
                    dchunk1[pl.ds(v * 16, 16)] = pv >> 16
                d1 = pltpu.async_copy(u_h.at[schunk1], rows1, sem1)
                d0.wait()
                pltpu.sync_copy(rows0, acc.at[dchunk0], add=True)
                d1.wait()
                pltpu.sync_copy(rows1, acc.at[dchunk1], add=True)
                return 0

            lax.fori_loop(0, nch // 2, db, 0)
            plsc.subcore_barrier()

            def co(j, _):
                pltpu.sync_copy(acc.at[pl.ds(s * 784 + j * 56, 56)],
                                rows0.at[pl.ds(0, 56)])
                pltpu.sync_copy(rows0.at[pl.ds(0, 56)],
                                out_h.at[pl.ds(lo + s * 784 + j * 56, 56)])
                return 0

            lax.fori_loop(0, 14, co, 0)
            plsc.subcore_barrier()

    return k(stash, counts, u)


def _tc_final(s2, u2p, dinv, batch2d, Wg2, bg2r, W1, b1r, W2, b2r, W3, b3r,
              W4, b4r):
    neg = float("-inf")

    def body(s2_ref, u2_ref, dv_ref, bt_ref, wg_ref, bg_ref, w1_ref, c1_ref,
             w2_ref, c2_ref, w3_ref, c3_ref, w4_ref, c4_ref, out_ref, acc):
        i = pl.program_id(0)

        @pl.when(i == 0)
        def _():
            acc[...] = jnp.full((G, 128), neg, jnp.float32)

        dv = dv_ref[...]
        h2 = jnp.dot(dv * (s2_ref[...] + u2_ref[...]), wg_ref[...],
                     preferred_element_type=jnp.float32)
        h2 = jnp.maximum(h2 + bg_ref[...], 0.0)
        row = i * BLK + lax.broadcasted_iota(jnp.int32, (BLK, 1), 0)
        hm = jnp.where(row < N, h2, neg)
        bt = bt_ref[...]
        g_lo = bt_ref[0, 0]
        g_hi = bt_ref[BLK - 1, 0]
        gi = lax.broadcasted_iota(jnp.int32, (G, 1), 0)

        def body_g(g, _):
            v = jnp.max(jnp.where(bt == g, hm, neg), axis=0, keepdims=True)
            acc[...] = jnp.where(gi == g, jnp.maximum(acc[...], v), acc[...])
            return 0

        lax.fori_loop(g_lo, g_hi + 1, body_g, 0)

        @pl.when(i == NBLK - 1)
        def _():
            h = acc[...]
            h = jnp.maximum(jnp.dot(h, w1_ref[...],
                                    preferred_element_type=jnp.float32)
                            + c1_ref[...], 0.0)
            h = jnp.maximum(jnp.dot(h, w2_ref[...],
                                    preferred_element_type=jnp.float32)
                            + c2_ref[...], 0.0)
            h = jnp.maximum(jnp.dot(h, w3_ref[...],
                                    preferred_element_type=jnp.float32)
                            + c3_ref[...], 0.0)
            h = jnp.maximum(jnp.dot(h, w4_ref[...],
                                    preferred_element_type=jnp.float32)
                            + c4_ref[...], 0.0)
            out_ref[...] = h

    full = lambda shape: pl.BlockSpec(shape, lambda i: (0, 0))
    return pl.pallas_call(
        body,
        grid=(NBLK,),
        in_specs=[
            pl.BlockSpec((BLK, 128), lambda i: (i, 0)),
            pl.BlockSpec((BLK, 128), lambda i: (i, 0)),
            pl.BlockSpec((BLK, 1), lambda i: (i, 0)),
            pl.BlockSpec((BLK, 1), lambda i: (i, 0)),
            full((128, 128)), full((1, 128)),
            full((128, 128)), full((1, 128)),
            full((128, 64)), full((1, 64)),
            full((64, 32)), full((1, 32)),
            full((32, 16)), full((1, 16)),
        ],
        out_specs=pl.BlockSpec((G, 16), lambda i: (0, 0)),
        out_shape=jax.ShapeDtypeStruct((G, 16), jnp.float32),
        scratch_shapes=[pltpu.VMEM((G, 128), jnp.float32)],
    )(s2, u2p, dinv, batch2d, Wg2, bg2r, W1, b1r, W2, b2r, W3, b3r, W4, b4r)


def kernel(x, edge_index, batch, Wg1, bg1, Wg2, bg2, W1, b1, W2, b2, W3, b3,
           W4, b4):
    f32 = jnp.float32
    i32 = jnp.int32
    src = jnp.concatenate([edge_index[0], jnp.zeros((EP - E,), i32)])
    dst = jnp.concatenate([edge_index[1], jnp.full((EP - E,), NP - 1, i32)])
    src2d = src.reshape(EP // 128, 128)
    dst2d = dst.reshape(EP // 128, 128)
    xp = jnp.zeros((NP, 128), f32).at[:N, :8].set(x)
    batch2d = jnp.concatenate([batch, jnp.full((NP - N,), G - 1, i32)])
    batch2d = batch2d.reshape(NP, 1)
    Wg1p = jnp.zeros((128, 128), f32).at[:8].set(Wg1)

    degp = _sc_deg(dst2d)
    stash, counts = _sc_compact(src, dst)
    dinv, u1p = _tc_prep(degp[:NP].reshape(NP, 1), degp[NP:].reshape(NP, 1),
                         xp)
    s1 = _sc_scatter(stash, counts, u1p)
    u2p = _tc_layer1(s1, u1p, dinv, Wg1p, bg1.reshape(1, 128))
    s2 = _sc_scatter(stash, counts, u2p)
    return _tc_final(s2, u2p, dinv, batch2d, Wg2, bg2.reshape(1, 128),
                     W1, b1.reshape(1, 128), W2, b2.reshape(1, 64),
                     W3, b3.reshape(1, 32), W4, b4.reshape(1, 16))
